# baseline (speedup 1.0000x reference)
import functools

import jax
import jax.numpy as jnp
from jax import lax
from jax.experimental import pallas as pl
from jax.experimental.pallas import tpu as pltpu
from jax.experimental.pallas import tpu_sc as plsc

ALPHA = 0.1
NC, NS = 2, 16


def _sc_mesh():
    return plsc.VectorSubcoreMesh(
        core_axis_name="c", subcore_axis_name="s", num_cores=NC, num_subcores=NS
    )


def _make_deg(N, E):
    ept = E // (NC * NS)
    K = 1000
    assert ept % K == 0 and N % K == 0
    nz = N // K

    @functools.partial(
        pl.kernel,
        mesh=_sc_mesh(),
        out_type=jax.ShapeDtypeStruct((NC * N,), jnp.float32),
        scratch_types=[
            pltpu.VMEM((K,), jnp.int32),
            pltpu.VMEM((K,), jnp.float32),
            pltpu.VMEM((K,), jnp.float32),
            pltpu.VMEM_SHARED((N,), jnp.float32),
        ],
    )
    def deg_k(dst_hbm, ones_hbm, zeros_hbm, out_hbm, idx_v, ones_v, zbuf, accd):
        c = lax.axis_index("c")
        s = lax.axis_index("s")
        wid = s * NC + c

        pltpu.sync_copy(zeros_hbm, zbuf)

        @pl.when(s < nz)
        def _():
            pltpu.sync_copy(zbuf, accd.at[pl.ds(s * K, K)])

        pltpu.sync_copy(ones_hbm, ones_v)
        plsc.subcore_barrier()

        def body(kk, carry):
            base = wid * ept + kk * K
            pltpu.sync_copy(dst_hbm.at[pl.ds(base, K)], idx_v)
            pltpu.sync_copy(ones_v, accd.at[idx_v], add=True)
            return carry

        lax.fori_loop(0, ept // K, body, 0)
        plsc.subcore_barrier()

        @pl.when(s < nz)
        def _():
            pltpu.sync_copy(accd.at[pl.ds(s * K, K)], zbuf)
            pltpu.sync_copy(zbuf, out_hbm.at[pl.ds(c * N + s * K, K)])

    return deg_k


def _make_prop(N, E, CH):
    ept = E // NS
    K = 80
    ZB = 80
    assert ept % K == 0 and N % ZB == 0
    ncH = ept // K
    nzb = N // ZB

    @functools.partial(
        pl.kernel,
        mesh=_sc_mesh(),
        out_type=jax.ShapeDtypeStruct((NC * N, CH), jnp.float32),
        scratch_types=[
            pltpu.VMEM((ept,), jnp.int32),
            pltpu.VMEM((ept,), jnp.int32),
            pltpu.VMEM((K, CH), jnp.float32),
            pltpu.VMEM((K, CH), jnp.float32),
            pltpu.VMEM((K, CH), jnp.float32),
            pltpu.SemaphoreType.DMA,
            pltpu.SemaphoreType.DMA,
            pltpu.SemaphoreType.DMA,
            pltpu.VMEM_SHARED((N, CH), jnp.float32),
        ],
    )
    def prop_k(y_hbm, srcidx_hbm, dst_hbm, zeros_hbm, out_hbm,
               idxs_v, idxd_v, rows0, rows1, rows2, sem0, sem1, sem2, acc):
        c = lax.axis_index("c")
        s = lax.axis_index("s")
        pltpu.sync_copy(srcidx_hbm.at[pl.ds(c * E + s * ept, ept)], idxs_v)
        pltpu.sync_copy(dst_hbm.at[pl.ds(s * ept, ept)], idxd_v)
        pltpu.sync_copy(zeros_hbm, rows0.at[pl.ds(0, ZB)])
        for j in range(nzb):
            @pl.when(s == j % NS)
            def _():
                pltpu.sync_copy(rows0.at[pl.ds(0, ZB)], acc.at[pl.ds(j * ZB, ZB)])
        plsc.subcore_barrier()

        def gather(kk, buf, sem):
            pltpu.async_copy(y_hbm.at[idxs_v.at[pl.ds(kk * K, K)]], buf, sem)

        def gwait(buf, sem):
            pltpu.make_async_copy(y_hbm.at[pl.ds(0, K)], buf, sem).wait()

        def scatter(kk, buf):
            pltpu.sync_copy(buf, acc.at[idxd_v.at[pl.ds(kk * K, K)]], add=True)

        gather(0, rows0, sem0)
        gather(1, rows1, sem1)

        def body(g, carry):
            k0 = 3 * g
            gather(k0 + 2, rows2, sem2)
            gwait(rows0, sem0)
            scatter(k0, rows0)
            gather(k0 + 3, rows0, sem0)
            gwait(rows1, sem1)
            scatter(k0 + 1, rows1)
            gather(k0 + 4, rows1, sem1)
            gwait(rows2, sem2)
            scatter(k0 + 2, rows2)
            return carry

        lax.fori_loop(0, (ncH - 2) // 3, body, 0)
        gwait(rows0, sem0)
        scatter(ncH - 2, rows0)
        gwait(rows1, sem1)
        scatter(ncH - 1, rows1)

        plsc.subcore_barrier()
        for j in range(nzb):
            @pl.when(s == j % NS)
            def _():
                pltpu.sync_copy(acc.at[pl.ds(j * ZB, ZB)], rows0.at[pl.ds(0, ZB)])
                pltpu.sync_copy(rows0.at[pl.ds(0, ZB)],
                                out_hbm.at[pl.ds(c * N + j * ZB, ZB)])

    return prop_k


def _stage_in(x, W_in, b_in, degp, bN):
    N, Cin = x.shape
    Cmid = W_in.shape[1]
    CH = Cmid // 2
    f32 = jnp.float32

    def body(x_ref, w_ref, b_ref, d_ref, xx_ref, y_ref, dinv_ref):
        xx = jnp.dot(x_ref[...], w_ref[...], preferred_element_type=f32)
        xx = jnp.maximum(xx + b_ref[...], 0.0)
        deg = d_ref[0] + d_ref[1] + 1.0
        dinv = lax.rsqrt(deg)
        y = xx * dinv
        xx_ref[...] = xx
        y_ref[0] = y[:, :CH]
        y_ref[1] = y[:, CH:]
        dinv_ref[...] = dinv

    return pl.pallas_call(
        body,
        grid=(N // bN,),
        in_specs=[
            pl.BlockSpec((bN, Cin), lambda i: (i, 0)),
            pl.BlockSpec((Cin, Cmid), lambda i: (0, 0)),
            pl.BlockSpec((1, Cmid), lambda i: (0, 0)),
            pl.BlockSpec((2, bN, 1), lambda i: (0, i, 0)),
        ],
        out_specs=[
            pl.BlockSpec((bN, Cmid), lambda i: (i, 0)),
            pl.BlockSpec((2, bN, CH), lambda i: (0, i, 0)),
            pl.BlockSpec((bN, 1), lambda i: (i, 0)),
        ],
        out_shape=[
            jax.ShapeDtypeStruct((N, Cmid), f32),
            jax.ShapeDtypeStruct((2, N, CH), f32),
            jax.ShapeDtypeStruct((N, 1), f32),
        ],
    )(x, W_in, b_in, degp)


def _stage_layer(acc2, y2, x0, dinv, W, g, be, residual, bN):
    N, Cmid = x0.shape
    CH = Cmid // 2
    f32 = jnp.float32

    def body(a_ref, y_ref, x0_ref, d_ref, w_ref, g_ref, be_ref, y_out):
        a = jnp.concatenate([a_ref[0], a_ref[1]], axis=1).astype(f32)
        y = jnp.concatenate([y_ref[0], y_ref[1]], axis=1).astype(f32)
        h = d_ref[...] * (a + y)
        h = (1.0 - ALPHA) * h + ALPHA * x0_ref[...]
        z = jnp.dot(h, w_ref[...], preferred_element_type=f32)
        mu = jnp.mean(z, axis=1, keepdims=True)
        zc = z - mu
        var = jnp.mean(zc * zc, axis=1, keepdims=True)
        t = zc * lax.rsqrt(var + 1e-5) * g_ref[...] + be_ref[...]
        if residual:
            t = t + x0_ref[...]
        t = jnp.maximum(t, 0.0)
        yn = t * d_ref[...]
        y_out[0] = yn[:, :CH]
        y_out[1] = yn[:, CH:]

    return pl.pallas_call(
        body,
        grid=(N // bN,),
        in_specs=[
            pl.BlockSpec((2, bN, CH), lambda i: (0, i, 0)),
            pl.BlockSpec((2, bN, CH), lambda i: (0, i, 0)),
            pl.BlockSpec((bN, Cmid), lambda i: (i, 0)),
            pl.BlockSpec((bN, 1), lambda i: (i, 0)),
            pl.BlockSpec((Cmid, Cmid), lambda i: (0, 0)),
            pl.BlockSpec((1, Cmid), lambda i: (0, 0)),
            pl.BlockSpec((1, Cmid), lambda i: (0, 0)),
        ],
        out_specs=pl.BlockSpec((2, bN, CH), lambda i: (0, i, 0)),
        out_shape=jax.ShapeDtypeStruct((2, N, CH), f32),
    )(acc2, y2, x0, dinv, W, g, be)


def _stage_last(acc2, y2, x0, dinv, W, g, be, W_out, b_out, bN):
    N, Cmid = x0.shape
    Cout = W_out.shape[1]
    f32 = jnp.float32

    def body(a_ref, y_ref, x0_ref, d_ref, w_ref, g_ref, be_ref,
             wo_ref, bo_ref, xx_ref, out_ref):
        a = jnp.concatenate([a_ref[0], a_ref[1]], axis=1).astype(f32)
        y = jnp.concatenate([y_ref[0], y_ref[1]], axis=1).astype(f32)
        h = d_ref[...] * (a + y)
        h = (1.0 - ALPHA) * h + ALPHA * x0_ref[...]
        z = jnp.dot(h, w_ref[...], preferred_element_type=f32)
        mu = jnp.mean(z, axis=1, keepdims=True)
        zc = z - mu
        var = jnp.mean(zc * zc, axis=1, keepdims=True)
        t = zc * lax.rsqrt(var + 1e-5) * g_ref[...] + be_ref[...]
        t = jnp.maximum(t, 0.0)
        xx_ref[...] = t
        out_ref[...] = jnp.dot(t, wo_ref[...], preferred_element_type=f32) + bo_ref[...]

    return pl.pallas_call(
        body,
        grid=(N // bN,),
        in_specs=[
            pl.BlockSpec((2, bN, Cmid // 2), lambda i: (0, i, 0)),
            pl.BlockSpec((2, bN, Cmid // 2), lambda i: (0, i, 0)),
            pl.BlockSpec((bN, Cmid), lambda i: (i, 0)),
            pl.BlockSpec((bN, 1), lambda i: (i, 0)),
            pl.BlockSpec((Cmid, Cmid), lambda i: (0, 0)),
            pl.BlockSpec((1, Cmid), lambda i: (0, 0)),
            pl.BlockSpec((1, Cmid), lambda i: (0, 0)),
            pl.BlockSpec((Cmid, Cout), lambda i: (0, 0)),
            pl.BlockSpec((1, Cout), lambda i: (0, 0)),
        ],
        out_specs=[
            pl.BlockSpec((bN, Cmid), lambda i: (i, 0)),
            pl.BlockSpec((bN, Cout), lambda i: (i, 0)),
        ],
        out_shape=[
            jax.ShapeDtypeStruct((N, Cmid), f32),
            jax.ShapeDtypeStruct((N, Cout), f32),
        ],
    )(acc2, y2, x0, dinv, W, g, be, W_out, b_out)


def kernel(x, edge, W_in, b_in, Wc0, Wc1, Wc2, g0, be0, g1, be1, g2, be2,
           W_out, b_out):
    N = x.shape[1]
    E = edge.shape[1]
    Cmid = W_in.shape[1]
    CH = Cmid // 2
    bN = 2000

    src = edge[0]
    dst = edge[1]
    srcidx2 = jnp.concatenate([src, src + N])

    ones_deg = jnp.ones((1000,), jnp.float32)
    zeros_deg = jnp.zeros((1000,), jnp.float32)
    zeros_acc = jnp.zeros((80, CH), jnp.float32)

    degp = _make_deg(N, E)(dst, ones_deg, zeros_deg)
    degp = degp.reshape(2, N, 1)

    xx0, y2, dinv = _stage_in(x[0], W_in, b_in.reshape(1, -1), degp, bN)

    prop = _make_prop(N, E, CH)
    convs = [Wc0, Wc1, Wc2]
    gammas = [g0.reshape(1, -1), g1.reshape(1, -1), g2.reshape(1, -1)]
    betas = [be0.reshape(1, -1), be1.reshape(1, -1), be2.reshape(1, -1)]

    for l in range(2):
        acc2 = prop(y2.reshape(2 * N, CH), srcidx2, dst, zeros_acc)
        acc2 = acc2.reshape(2, N, CH)
        y2 = _stage_layer(acc2, y2, xx0, dinv, convs[l], gammas[l], betas[l],
                          residual=(l == 1), bN=bN)

    acc2 = prop(y2.reshape(2 * N, CH), srcidx2, dst, zeros_acc)
    acc2 = acc2.reshape(2, N, CH)
    xx3, out = _stage_last(acc2, y2, xx0, dinv, convs[2], gammas[2], betas[2],
                           W_out, b_out.reshape(1, -1), bN)
    return (xx3[None, :, :], out[None, :, :])


# speedup vs baseline: 1.0119x; 1.0119x over previous
import functools

import jax
import jax.numpy as jnp
from jax import lax
from jax.experimental import pallas as pl
from jax.experimental.pallas import tpu as pltpu
from jax.experimental.pallas import tpu_sc as plsc

ALPHA = 0.1
NC, NS = 2, 16


def _sc_mesh():
    return plsc.VectorSubcoreMesh(
        core_axis_name="c", subcore_axis_name="s", num_cores=NC, num_subcores=NS
    )


def _make_deg(N, E):
    ept = E // (NC * NS)
    K = 1000
    assert ept % K == 0 and N % K == 0
    nz = N // K

    @functools.partial(
        pl.kernel,
        mesh=_sc_mesh(),
        out_type=jax.ShapeDtypeStruct((NC * N,), jnp.float32),
        scratch_types=[
            pltpu.VMEM((K,), jnp.int32),
            pltpu.VMEM((K,), jnp.float32),
            pltpu.VMEM((K,), jnp.float32),
            pltpu.VMEM_SHARED((N,), jnp.float32),
        ],
    )
    def deg_k(dst_hbm, ones_hbm, zeros_hbm, out_hbm, idx_v, ones_v, zbuf, accd):
        c = lax.axis_index("c")
        s = lax.axis_index("s")
        wid = s * NC + c

        pltpu.sync_copy(zeros_hbm, zbuf)

        @pl.when(s < nz)
        def _():
            pltpu.sync_copy(zbuf, accd.at[pl.ds(s * K, K)])

        pltpu.sync_copy(ones_hbm, ones_v)
        plsc.subcore_barrier()

        def body(kk, carry):
            base = wid * ept + kk * K
            pltpu.sync_copy(dst_hbm.at[pl.ds(base, K)], idx_v)
            pltpu.sync_copy(ones_v, accd.at[idx_v], add=True)
            return carry

        lax.fori_loop(0, ept // K, body, 0)
        plsc.subcore_barrier()

        @pl.when(s < nz)
        def _():
            pltpu.sync_copy(accd.at[pl.ds(s * K, K)], zbuf)
            pltpu.sync_copy(zbuf, out_hbm.at[pl.ds(c * N + s * K, K)])

    return deg_k


def _make_prop(N, E, CH):
    ept = E // NS
    K = 80
    ZB = 80
    assert ept % K == 0 and N % ZB == 0
    ncH = ept // K
    nzb = N // ZB

    @functools.partial(
        pl.kernel,
        mesh=_sc_mesh(),
        out_type=jax.ShapeDtypeStruct((NC * N, CH), jnp.float32),
        scratch_types=[
            pltpu.VMEM((ept,), jnp.int32),
            pltpu.VMEM((ept,), jnp.int32),
            pltpu.VMEM((K, CH), jnp.float32),
            pltpu.VMEM((K, CH), jnp.float32),
            pltpu.VMEM((K, CH), jnp.float32),
            pltpu.SemaphoreType.DMA,
            pltpu.SemaphoreType.DMA,
            pltpu.SemaphoreType.DMA,
            pltpu.VMEM_SHARED((N, CH), jnp.float32),
        ],
    )
    def prop_k(y_hbm, srcidx_hbm, dst_hbm, zeros_hbm, out_hbm,
               idxs_v, idxd_v, rows0, rows1, rows2, sem0, sem1, sem2, acc):
        c = lax.axis_index("c")
        s = lax.axis_index("s")
        pltpu.sync_copy(srcidx_hbm.at[pl.ds(c * E + s * ept, ept)], idxs_v)
        pltpu.sync_copy(dst_hbm.at[pl.ds(s * ept, ept)], idxd_v)

        def gather(kk, buf, sem):
            pltpu.async_copy(y_hbm.at[idxs_v.at[pl.ds(kk * K, K)]], buf, sem)

        def gwait(buf, sem):
            pltpu.make_async_copy(y_hbm.at[pl.ds(0, K)], buf, sem).wait()

        def scatter(kk, buf):
            pltpu.sync_copy(buf, acc.at[idxd_v.at[pl.ds(kk * K, K)]], add=True)

        gather(0, rows0, sem0)
        gather(1, rows1, sem1)
        pltpu.sync_copy(zeros_hbm, rows2.at[pl.ds(0, ZB)])
        for j in range(nzb):
            @pl.when(s == j % NS)
            def _():
                pltpu.async_copy(rows2.at[pl.ds(0, ZB)],
                                 acc.at[pl.ds(j * ZB, ZB)], sem2)
        for j in range(nzb):
            @pl.when(s == j % NS)
            def _():
                pltpu.make_async_copy(
                    rows2.at[pl.ds(0, ZB)], acc.at[pl.ds(j * ZB, ZB)], sem2
                ).wait()
        plsc.subcore_barrier()

        def body(g, carry):
            k0 = 3 * g
            gather(k0 + 2, rows2, sem2)
            gwait(rows0, sem0)
            scatter(k0, rows0)
            gather(k0 + 3, rows0, sem0)
            gwait(rows1, sem1)
            scatter(k0 + 1, rows1)
            gather(k0 + 4, rows1, sem1)
            gwait(rows2, sem2)
            scatter(k0 + 2, rows2)
            return carry

        lax.fori_loop(0, (ncH - 2) // 3, body, 0)
        gwait(rows0, sem0)
        scatter(ncH - 2, rows0)
        gwait(rows1, sem1)
        scatter(ncH - 1, rows1)

        plsc.subcore_barrier()
        wbuf = (rows0, rows1)
        wsem = (sem0, sem1)
        for j in range(nzb):
            b = (j // NS) % 2
            @pl.when(s == j % NS)
            def _():
                if j // NS >= 2:
                    pltpu.make_async_copy(
                        wbuf[b].at[pl.ds(0, ZB)],
                        out_hbm.at[pl.ds(c * N + (j - 2 * NS) * ZB, ZB)],
                        wsem[b]).wait()
                pltpu.sync_copy(acc.at[pl.ds(j * ZB, ZB)], wbuf[b].at[pl.ds(0, ZB)])
                pltpu.async_copy(wbuf[b].at[pl.ds(0, ZB)],
                                 out_hbm.at[pl.ds(c * N + j * ZB, ZB)], wsem[b])
        for j in range(max(0, nzb - 2 * NS), nzb):
            b = (j // NS) % 2
            @pl.when(s == j % NS)
            def _():
                pltpu.make_async_copy(
                    wbuf[b].at[pl.ds(0, ZB)],
                    out_hbm.at[pl.ds(c * N + j * ZB, ZB)], wsem[b]).wait()

    return prop_k


def _stage_in(x, W_in, b_in, degp, bN):
    N, Cin = x.shape
    Cmid = W_in.shape[1]
    CH = Cmid // 2
    f32 = jnp.float32

    def body(x_ref, w_ref, b_ref, d_ref, xx_ref, y_ref, dinv_ref):
        xx = jnp.dot(x_ref[...], w_ref[...], preferred_element_type=f32)
        xx = jnp.maximum(xx + b_ref[...], 0.0)
        deg = d_ref[0] + d_ref[1] + 1.0
        dinv = lax.rsqrt(deg)
        y = xx * dinv
        xx_ref[...] = xx
        y_ref[0] = y[:, :CH]
        y_ref[1] = y[:, CH:]
        dinv_ref[...] = dinv

    return pl.pallas_call(
        body,
        grid=(N // bN,),
        in_specs=[
            pl.BlockSpec((bN, Cin), lambda i: (i, 0)),
            pl.BlockSpec((Cin, Cmid), lambda i: (0, 0)),
            pl.BlockSpec((1, Cmid), lambda i: (0, 0)),
            pl.BlockSpec((2, bN, 1), lambda i: (0, i, 0)),
        ],
        out_specs=[
            pl.BlockSpec((bN, Cmid), lambda i: (i, 0)),
            pl.BlockSpec((2, bN, CH), lambda i: (0, i, 0)),
            pl.BlockSpec((bN, 1), lambda i: (i, 0)),
        ],
        out_shape=[
            jax.ShapeDtypeStruct((N, Cmid), f32),
            jax.ShapeDtypeStruct((2, N, CH), f32),
            jax.ShapeDtypeStruct((N, 1), f32),
        ],
    )(x, W_in, b_in, degp)


def _stage_layer(acc2, y2, x0, dinv, W, g, be, residual, bN):
    N, Cmid = x0.shape
    CH = Cmid // 2
    f32 = jnp.float32

    def body(a_ref, y_ref, x0_ref, d_ref, w_ref, g_ref, be_ref, y_out):
        a = jnp.concatenate([a_ref[0], a_ref[1]], axis=1).astype(f32)
        y = jnp.concatenate([y_ref[0], y_ref[1]], axis=1).astype(f32)
        h = d_ref[...] * (a + y)
        h = (1.0 - ALPHA) * h + ALPHA * x0_ref[...]
        z = jnp.dot(h, w_ref[...], preferred_element_type=f32)
        mu = jnp.mean(z, axis=1, keepdims=True)
        zc = z - mu
        var = jnp.mean(zc * zc, axis=1, keepdims=True)
        t = zc * lax.rsqrt(var + 1e-5) * g_ref[...] + be_ref[...]
        if residual:
            t = t + x0_ref[...]
        t = jnp.maximum(t, 0.0)
        yn = t * d_ref[...]
        y_out[0] = yn[:, :CH]
        y_out[1] = yn[:, CH:]

    return pl.pallas_call(
        body,
        grid=(N // bN,),
        in_specs=[
            pl.BlockSpec((2, bN, CH), lambda i: (0, i, 0)),
            pl.BlockSpec((2, bN, CH), lambda i: (0, i, 0)),
            pl.BlockSpec((bN, Cmid), lambda i: (i, 0)),
            pl.BlockSpec((bN, 1), lambda i: (i, 0)),
            pl.BlockSpec((Cmid, Cmid), lambda i: (0, 0)),
            pl.BlockSpec((1, Cmid), lambda i: (0, 0)),
            pl.BlockSpec((1, Cmid), lambda i: (0, 0)),
        ],
        out_specs=pl.BlockSpec((2, bN, CH), lambda i: (0, i, 0)),
        out_shape=jax.ShapeDtypeStruct((2, N, CH), f32),
    )(acc2, y2, x0, dinv, W, g, be)


def _stage_last(acc2, y2, x0, dinv, W, g, be, W_out, b_out, bN):
    N, Cmid = x0.shape
    Cout = W_out.shape[1]
    f32 = jnp.float32

    def body(a_ref, y_ref, x0_ref, d_ref, w_ref, g_ref, be_ref,
             wo_ref, bo_ref, xx_ref, out_ref):
        a = jnp.concatenate([a_ref[0], a_ref[1]], axis=1).astype(f32)
        y = jnp.concatenate([y_ref[0], y_ref[1]], axis=1).astype(f32)
        h = d_ref[...] * (a + y)
        h = (1.0 - ALPHA) * h + ALPHA * x0_ref[...]
        z = jnp.dot(h, w_ref[...], preferred_element_type=f32)
        mu = jnp.mean(z, axis=1, keepdims=True)
        zc = z - mu
        var = jnp.mean(zc * zc, axis=1, keepdims=True)
        t = zc * lax.rsqrt(var + 1e-5) * g_ref[...] + be_ref[...]
        t = jnp.maximum(t, 0.0)
        xx_ref[...] = t
        out_ref[...] = jnp.dot(t, wo_ref[...], preferred_element_type=f32) + bo_ref[...]

    return pl.pallas_call(
        body,
        grid=(N // bN,),
        in_specs=[
            pl.BlockSpec((2, bN, Cmid // 2), lambda i: (0, i, 0)),
            pl.BlockSpec((2, bN, Cmid // 2), lambda i: (0, i, 0)),
            pl.BlockSpec((bN, Cmid), lambda i: (i, 0)),
            pl.BlockSpec((bN, 1), lambda i: (i, 0)),
            pl.BlockSpec((Cmid, Cmid), lambda i: (0, 0)),
            pl.BlockSpec((1, Cmid), lambda i: (0, 0)),
            pl.BlockSpec((1, Cmid), lambda i: (0, 0)),
            pl.BlockSpec((Cmid, Cout), lambda i: (0, 0)),
            pl.BlockSpec((1, Cout), lambda i: (0, 0)),
        ],
        out_specs=[
            pl.BlockSpec((bN, Cmid), lambda i: (i, 0)),
            pl.BlockSpec((bN, Cout), lambda i: (i, 0)),
        ],
        out_shape=[
            jax.ShapeDtypeStruct((N, Cmid), f32),
            jax.ShapeDtypeStruct((N, Cout), f32),
        ],
    )(acc2, y2, x0, dinv, W, g, be, W_out, b_out)


def kernel(x, edge, W_in, b_in, Wc0, Wc1, Wc2, g0, be0, g1, be1, g2, be2,
           W_out, b_out):
    N = x.shape[1]
    E = edge.shape[1]
    Cmid = W_in.shape[1]
    CH = Cmid // 2
    bN = 1000

    src = edge[0]
    dst = edge[1]
    srcidx2 = jnp.concatenate([src, src + N])

    ones_deg = jnp.ones((1000,), jnp.float32)
    zeros_deg = jnp.zeros((1000,), jnp.float32)
    zeros_acc = jnp.zeros((80, CH), jnp.float32)

    degp = _make_deg(N, E)(dst, ones_deg, zeros_deg)
    degp = degp.reshape(2, N, 1)

    xx0, y2, dinv = _stage_in(x[0], W_in, b_in.reshape(1, -1), degp, bN)

    prop = _make_prop(N, E, CH)
    convs = [Wc0, Wc1, Wc2]
    gammas = [g0.reshape(1, -1), g1.reshape(1, -1), g2.reshape(1, -1)]
    betas = [be0.reshape(1, -1), be1.reshape(1, -1), be2.reshape(1, -1)]

    for l in range(2):
        acc2 = prop(y2.reshape(2 * N, CH), srcidx2, dst, zeros_acc)
        acc2 = acc2.reshape(2, N, CH)
        y2 = _stage_layer(acc2, y2, xx0, dinv, convs[l], gammas[l], betas[l],
                          residual=(l == 1), bN=bN)

    acc2 = prop(y2.reshape(2 * N, CH), srcidx2, dst, zeros_acc)
    acc2 = acc2.reshape(2, N, CH)
    xx3, out = _stage_last(acc2, y2, xx0, dinv, convs[2], gammas[2], betas[2],
                           W_out, b_out.reshape(1, -1), bN)
    return (xx3[None, :, :], out[None, :, :])
